# baseline (speedup 1.0000x reference)
import dataclasses
import functools

import jax
import jax.numpy as jnp
from jax import lax
from jax.experimental import pallas as pl
from jax.experimental.pallas import tpu as pltpu
from jax.experimental.pallas import tpu_sc as plsc

N = 10000
D = 128
DE = 16
E = 320000
EP = 327680
WIN = 128
NG = EP // WIN
NC = 2
NS = 16
NW = NC * NS
WPT = NG // NW
EPT = EP // NW
NPAD = 10240
RPT = NPAD // NS


def _sc_params():
    cp = pltpu.CompilerParams()
    if "needs_layout_passes" in pltpu.CompilerParams.__dataclass_fields__:
        cp = dataclasses.replace(cp, needs_layout_passes=False)
    return cp


def _silu(x):
    return x * jax.nn.sigmoid(x)


def _tc_project(h, w1a, w1b):
    blk = 1000

    def body(h_ref, wa_ref, wb_ref, hp_ref, hq_ref):
        hh = h_ref[...]
        hp_ref[...] = jnp.dot(hh, wa_ref[...], preferred_element_type=jnp.float32)
        hq_ref[...] = jnp.dot(hh, wb_ref[...], preferred_element_type=jnp.float32)

    return pl.pallas_call(
        body,
        grid=(N // blk,),
        in_specs=[
            pl.BlockSpec((blk, D), lambda i: (i, 0)),
            pl.BlockSpec((D, D), lambda i: (0, 0)),
            pl.BlockSpec((D, D), lambda i: (0, 0)),
        ],
        out_specs=[
            pl.BlockSpec((blk, D), lambda i: (i, 0)),
            pl.BlockSpec((blk, D), lambda i: (i, 0)),
        ],
        out_shape=[jax.ShapeDtypeStruct((N, D), jnp.float32)] * 2,
    )(h, w1a, w1b)


def _sc_gather(hp, hq, cx, cy, cz, g_row, g_col):
    mesh = plsc.VectorSubcoreMesh(core_axis_name="c", subcore_axis_name="s")
    out_type = (
        jax.ShapeDtypeStruct((EP, D), jnp.float32),
        jax.ShapeDtypeStruct((EP, D), jnp.float32),
        jax.ShapeDtypeStruct((EP,), jnp.float32),
        jax.ShapeDtypeStruct((EP,), jnp.float32),
        jax.ShapeDtypeStruct((EP,), jnp.float32),
        jax.ShapeDtypeStruct((EP,), jnp.float32),
    )

    @functools.partial(
        pl.kernel,
        out_type=out_type,
        mesh=mesh,
        scratch_types=[
            pltpu.VMEM((N,), jnp.float32),
            pltpu.VMEM((N,), jnp.float32),
            pltpu.VMEM((N,), jnp.float32),
            pltpu.VMEM((WPT, WIN), jnp.int32),
            pltpu.VMEM((WPT, WIN), jnp.int32),
            pltpu.VMEM((WIN, D), jnp.float32),
            pltpu.VMEM((WIN, D), jnp.float32),
            pltpu.VMEM((WIN,), jnp.float32),
            pltpu.VMEM((WIN,), jnp.float32),
            pltpu.VMEM((WIN,), jnp.float32),
            pltpu.VMEM((WIN,), jnp.float32),
            pltpu.SemaphoreType.DMA,
            pltpu.SemaphoreType.DMA,
        ],
        compiler_params=_sc_params(),
    )
    def k(hp_h, hq_h, cx_h, cy_h, cz_h, gr_h, gc_h,
          a_h, b_h, rad_h, dx_h, dy_h, dz_h,
          cx_v, cy_v, cz_v, gr_v, gc_v, a_v, b_v,
          rad_v, dx_v, dy_v, dz_v, semA, semB):
        c = lax.axis_index("c")
        s = lax.axis_index("s")
        w = s * NC + c
        base_g = w * WPT
        pltpu.sync_copy(cx_h, cx_v)
        pltpu.sync_copy(cy_h, cy_v)
        pltpu.sync_copy(cz_h, cz_v)
        pltpu.sync_copy(gr_h.at[pl.ds(base_g, WPT)], gr_v)
        pltpu.sync_copy(gc_h.at[pl.ds(base_g, WPT)], gc_v)

        @pl.loop(0, WPT)
        def _(j):
            e0 = (base_g + j) * WIN
            cpA = pltpu.async_copy(hp_h.at[gr_v.at[j]], a_v, semA)
            cpB = pltpu.async_copy(hq_h.at[gc_v.at[j]], b_v, semB)
            for g in range(WIN // 16):
                ir = gr_v[j, pl.ds(g * 16, 16)]
                ic = gc_v[j, pl.ds(g * 16, 16)]
                dx = plsc.load_gather(cx_v, [ir]) - plsc.load_gather(cx_v, [ic])
                dy = plsc.load_gather(cy_v, [ir]) - plsc.load_gather(cy_v, [ic])
                dz = plsc.load_gather(cz_v, [ir]) - plsc.load_gather(cz_v, [ic])
                dx_v[pl.ds(g * 16, 16)] = dx
                dy_v[pl.ds(g * 16, 16)] = dy
                dz_v[pl.ds(g * 16, 16)] = dz
                rad_v[pl.ds(g * 16, 16)] = dx * dx + dy * dy + dz * dz
            pltpu.sync_copy(rad_v, rad_h.at[pl.ds(e0, WIN)])
            pltpu.sync_copy(dx_v, dx_h.at[pl.ds(e0, WIN)])
            pltpu.sync_copy(dy_v, dy_h.at[pl.ds(e0, WIN)])
            pltpu.sync_copy(dz_v, dz_h.at[pl.ds(e0, WIN)])
            cpA.wait()
            cpB.wait()
            pltpu.sync_copy(a_v, a_h.at[pl.ds(e0, WIN)])
            pltpu.sync_copy(b_v, b_h.at[pl.ds(e0, WIN)])

    return k(hp, hq, cx, cy, cz, g_row, g_col)


def _tc_edge_mlp(a, b, rad, dx, dy, dz, ea,
                 w_rad, w1e, b1, w2, b2, cw1, cb1, cw2, cb2):
    blk = 512

    def body(a_ref, b_ref, rad_ref, dx_ref, dy_ref, dz_ref, ea_ref,
             wr_ref, w1e_ref, b1_ref, w2_ref, b2_ref, cw1_ref, cb1_ref,
             cw2_ref, cb2_ref, m_ref, dxw_ref, dyw_ref, dzw_ref):
        rad_blk = rad_ref[...]
        x = (a_ref[...] + b_ref[...] + rad_blk * wr_ref[...]
             + jnp.dot(ea_ref[...], w1e_ref[...], preferred_element_type=jnp.float32)
             + b1_ref[...])
        t = _silu(x).astype(jnp.bfloat16)
        m = _silu(jnp.dot(t, w2_ref[...].astype(jnp.bfloat16),
                          preferred_element_type=jnp.float32)
                  + b2_ref[...])
        m_ref[...] = m
        u = _silu(jnp.dot(m.astype(jnp.bfloat16),
                          cw1_ref[...].astype(jnp.bfloat16),
                          preferred_element_type=jnp.float32)
                  + cb1_ref[...])
        wc = jnp.dot(u, cw2_ref[...], preferred_element_type=jnp.float32) + cb2_ref[...]
        wcn = wc / (jnp.sqrt(rad_blk) + 1e-8)
        dxw_ref[...] = dx_ref[...] * wcn
        dyw_ref[...] = dy_ref[...] * wcn
        dzw_ref[...] = dz_ref[...] * wcn

    full = lambda r, c: pl.BlockSpec((r, c), lambda i: (0, 0))
    col = pl.BlockSpec((blk, 1), lambda i: (i, 0))
    return pl.pallas_call(
        body,
        grid=(EP // blk,),
        in_specs=[
            pl.BlockSpec((blk, D), lambda i: (i, 0)),
            pl.BlockSpec((blk, D), lambda i: (i, 0)),
            col, col, col, col,
            pl.BlockSpec((blk, DE), lambda i: (i, 0)),
            full(1, D), full(DE, D), full(1, D), full(D, D), full(1, D),
            full(D, D), full(1, D), full(D, 1), full(1, 1),
        ],
        out_specs=[pl.BlockSpec((blk, D), lambda i: (i, 0)), col, col, col],
        out_shape=[
            jax.ShapeDtypeStruct((EP, D), jnp.float32),
            jax.ShapeDtypeStruct((EP, 1), jnp.float32),
            jax.ShapeDtypeStruct((EP, 1), jnp.float32),
            jax.ShapeDtypeStruct((EP, 1), jnp.float32),
        ],
    )(a, b, rad, dx, dy, dz, ea, w_rad, w1e, b1, w2, b2, cw1, cb1, cw2, cb2)


def _sc_scatter_m(m, s_row):
    mesh = plsc.VectorSubcoreMesh(core_axis_name="c", subcore_axis_name="s")
    out_type = jax.ShapeDtypeStruct((NC, NPAD, D), jnp.float32)

    @functools.partial(
        pl.kernel,
        out_type=out_type,
        mesh=mesh,
        scratch_types=[
            pltpu.VMEM((1, WIN), jnp.int32),
            pltpu.VMEM((WIN, D), jnp.float32),
            pltpu.VMEM_SHARED((NPAD, D), jnp.float32),
            pltpu.SemaphoreType.DMA,
        ],
        compiler_params=_sc_params(),
    )
    def k(m_h, sr_h, mo_h, sr_v, m_v, macc, sem):
        c = lax.axis_index("c")
        s = lax.axis_index("s")
        w = s * NC + c
        base_g = w * WPT
        zero16 = jnp.zeros((16,), jnp.float32)

        @pl.loop(0, WIN)
        def _(i):
            for g in range(D // 16):
                m_v[i, pl.ds(g * 16, 16)] = zero16

        row0 = s * RPT
        for kb in range(RPT // WIN):
            pltpu.sync_copy(m_v, macc.at[pl.ds(row0 + kb * WIN, WIN)])
        plsc.subcore_barrier()

        @pl.loop(0, WPT)
        def _(j):
            gj = base_g + j
            pltpu.sync_copy(sr_h.at[pl.ds(gj, 1)], sr_v)
            pltpu.sync_copy(m_h.at[pl.ds(gj * WIN, WIN)], m_v)
            pltpu.sync_copy(m_v, macc.at[sr_v.at[0]], add=True)

        plsc.subcore_barrier()
        pltpu.sync_copy(macc.at[pl.ds(row0, RPT)], mo_h.at[c].at[pl.ds(row0, RPT)])

    return k(m, s_row)


def _sc_scatter_c(sr, dxw, dyw, dzw, z4):
    mesh = plsc.VectorSubcoreMesh(core_axis_name="c", subcore_axis_name="s")
    out_type = jax.ShapeDtypeStruct((NW, NPAD * 4), jnp.float32)

    @functools.partial(
        pl.kernel,
        out_type=out_type,
        mesh=mesh,
        scratch_types=[
            pltpu.VMEM((EPT,), jnp.int32),
            pltpu.VMEM((EPT,), jnp.float32),
            pltpu.VMEM((EPT,), jnp.float32),
            pltpu.VMEM((EPT,), jnp.float32),
            pltpu.VMEM((NPAD * 4,), jnp.float32),
        ],
        compiler_params=_sc_params(),
    )
    def k(sr_h, dx_h, dy_h, dz_h, z4_h, co_h, sr_v, dx_v, dy_v, dz_v, acc_v):
        c = lax.axis_index("c")
        s = lax.axis_index("s")
        w = s * NC + c
        e0 = w * EPT
        pltpu.sync_copy(z4_h, acc_v)
        pltpu.sync_copy(sr_h.at[pl.ds(e0, EPT)], sr_v)
        pltpu.sync_copy(dx_h.at[pl.ds(e0, EPT)], dx_v)
        pltpu.sync_copy(dy_h.at[pl.ds(e0, EPT)], dy_v)
        pltpu.sync_copy(dz_h.at[pl.ds(e0, EPT)], dz_v)
        @pl.loop(0, EPT // 16)
        def _(i):
            sl = pl.ds(i * 16, 16)
            idx = sr_v[sl] * 4
            plsc.addupdate_scatter(acc_v, [idx], dx_v[sl])
            plsc.addupdate_scatter(acc_v, [idx + 1], dy_v[sl])
            plsc.addupdate_scatter(acc_v, [idx + 2], dz_v[sl])

        pltpu.sync_copy(acc_v, co_h.at[w])

    return k(sr, dxw, dyw, dzw, z4)


def _tc_node(h, mparts, cparts, coord4, vel4,
             vw1, vb1, vw2, vb2, nw1a, nw1b, nb1, nw2, nb2):
    blk = 1000

    def body(h_ref, mp_ref, cp_ref, c4_ref, v4_ref, vw1_ref, vb1_ref, vw2_ref,
             vb2_ref, na_ref, nbm_ref, nb1_ref, nw2_ref, nb2_ref, ho_ref, co_ref):
        hh = h_ref[...]
        m_i = mp_ref[0] + mp_ref[1]
        csum = jnp.sum(cp_ref[...], axis=0)
        u = _silu(jnp.dot(hh, vw1_ref[...], preferred_element_type=jnp.float32)
                  + vb1_ref[...])
        vw = jnp.dot(u, vw2_ref[...], preferred_element_type=jnp.float32) + vb2_ref[...]
        co_ref[...] = c4_ref[...] + csum + v4_ref[...] * vw
        t = _silu(jnp.dot(hh, na_ref[...], preferred_element_type=jnp.float32)
                  + jnp.dot(m_i, nbm_ref[...], preferred_element_type=jnp.float32)
                  + nb1_ref[...])
        ho_ref[...] = jnp.dot(t, nw2_ref[...], preferred_element_type=jnp.float32) + nb2_ref[...]

    full = lambda r, c: pl.BlockSpec((r, c), lambda i: (0, 0))
    return pl.pallas_call(
        body,
        grid=(N // blk,),
        in_specs=[
            pl.BlockSpec((blk, D), lambda i: (i, 0)),
            pl.BlockSpec((NC, blk, D), lambda i: (0, i, 0)),
            pl.BlockSpec((NW, blk, 4), lambda i: (0, i, 0)),
            pl.BlockSpec((blk, 4), lambda i: (i, 0)),
            pl.BlockSpec((blk, 4), lambda i: (i, 0)),
            full(D, D), full(1, D), full(D, 1), full(1, 1),
            full(D, D), full(D, D), full(1, D), full(D, D), full(1, D),
        ],
        out_specs=[
            pl.BlockSpec((blk, D), lambda i: (i, 0)),
            pl.BlockSpec((blk, 4), lambda i: (i, 0)),
        ],
        out_shape=[
            jax.ShapeDtypeStruct((N, D), jnp.float32),
            jax.ShapeDtypeStruct((N, 4), jnp.float32),
        ],
    )(h, mparts, cparts, coord4, vel4, vw1, vb1, vw2, vb2, nw1a, nw1b, nb1, nw2, nb2)


def kernel(h, edge_index, coord, vel, edge_attr,
           edge_w1, edge_b1, edge_w2, edge_b2,
           coord_w1, coord_b1, coord_w2, coord_b2,
           vel_w1, vel_b1, vel_w2, vel_b2,
           node_w1, node_b1, node_w2, node_b2):
    w1a = edge_w1[:D]
    w1b = edge_w1[D:2 * D]
    w_rad = edge_w1[2 * D].reshape(1, D)
    w1e = edge_w1[2 * D + 1:]

    row = edge_index[0].astype(jnp.int32)
    col = edge_index[1].astype(jnp.int32)
    pad = EP - E
    g_row = jnp.concatenate([row, jnp.zeros((pad,), jnp.int32)]).reshape(NG, WIN)
    g_col = jnp.concatenate([col, jnp.zeros((pad,), jnp.int32)]).reshape(NG, WIN)
    s_row = jnp.concatenate([row, jnp.full((pad,), N, jnp.int32)]).reshape(NG, WIN)
    ea = jnp.concatenate([edge_attr, jnp.zeros((pad, DE), jnp.float32)], axis=0)

    cx = coord[:, 0] + 0.0
    cy = coord[:, 1] + 0.0
    cz = coord[:, 2] + 0.0
    coord4 = jnp.pad(coord, ((0, 0), (0, 1)))
    vel4 = jnp.pad(vel, ((0, 0), (0, 1)))
    z4 = jnp.zeros((NPAD * 4,), jnp.float32)

    hp, hq = _tc_project(h, w1a, w1b)
    a, b, rad, dx, dy, dz = _sc_gather(hp, hq, cx, cy, cz, g_row, g_col)
    m_ij, dxw, dyw, dzw = _tc_edge_mlp(
        a, b, rad.reshape(EP, 1), dx.reshape(EP, 1), dy.reshape(EP, 1),
        dz.reshape(EP, 1), ea, w_rad, w1e, edge_b1.reshape(1, D),
        edge_w2, edge_b2.reshape(1, D), coord_w1, coord_b1.reshape(1, D),
        coord_w2, coord_b2.reshape(1, 1))
    mparts = _sc_scatter_m(m_ij, s_row)
    cparts = _sc_scatter_c(s_row.reshape(EP), dxw.reshape(EP),
                           dyw.reshape(EP), dzw.reshape(EP), z4)
    cparts = cparts.reshape(NW, NPAD, 4)
    h_out, co4 = _tc_node(
        h, mparts, cparts, coord4, vel4,
        vel_w1, vel_b1.reshape(1, D), vel_w2, vel_b2.reshape(1, 1),
        node_w1[:D], node_w1[D:], node_b1.reshape(1, D),
        node_w2, node_b2.reshape(1, D))
    return (h_out, co4[:, :3])


# speedup vs baseline: 1.0616x; 1.0616x over previous
import dataclasses
import functools

import jax
import jax.numpy as jnp
from jax import lax
from jax.experimental import pallas as pl
from jax.experimental.pallas import tpu as pltpu
from jax.experimental.pallas import tpu_sc as plsc

N = 10000
D = 128
DE = 16
E = 320000
EP = 327680
WIN = 128
NG = EP // WIN
NC = 2
NS = 16
NW = NC * NS
WPT = NG // NW
EPT = EP // NW
NPAD = 10240
RPT = NPAD // NS


def _sc_params():
    cp = pltpu.CompilerParams()
    if "needs_layout_passes" in pltpu.CompilerParams.__dataclass_fields__:
        cp = dataclasses.replace(cp, needs_layout_passes=False)
    return cp


def _silu(x):
    return x * jax.nn.sigmoid(x)


def _tc_project(h, w1a, w1b):
    blk = 1000

    def body(h_ref, wa_ref, wb_ref, hp_ref, hq_ref):
        hh = h_ref[...]
        hp_ref[...] = jnp.dot(hh, wa_ref[...], preferred_element_type=jnp.float32)
        hq_ref[...] = jnp.dot(hh, wb_ref[...], preferred_element_type=jnp.float32)

    return pl.pallas_call(
        body,
        grid=(N // blk,),
        in_specs=[
            pl.BlockSpec((blk, D), lambda i: (i, 0)),
            pl.BlockSpec((D, D), lambda i: (0, 0)),
            pl.BlockSpec((D, D), lambda i: (0, 0)),
        ],
        out_specs=[
            pl.BlockSpec((blk, D), lambda i: (i, 0)),
            pl.BlockSpec((blk, D), lambda i: (i, 0)),
        ],
        out_shape=[jax.ShapeDtypeStruct((N, D), jnp.float32)] * 2,
    )(h, w1a, w1b)


def _sc_gather(hp, hq, cx, cy, cz, g_row, g_col):
    mesh = plsc.VectorSubcoreMesh(core_axis_name="c", subcore_axis_name="s")
    out_type = (
        jax.ShapeDtypeStruct((EP, D), jnp.float32),
        jax.ShapeDtypeStruct((EP, D), jnp.float32),
        jax.ShapeDtypeStruct((EP,), jnp.float32),
        jax.ShapeDtypeStruct((EP,), jnp.float32),
        jax.ShapeDtypeStruct((EP,), jnp.float32),
        jax.ShapeDtypeStruct((EP,), jnp.float32),
    )

    @functools.partial(
        pl.kernel,
        out_type=out_type,
        mesh=mesh,
        scratch_types=[
            pltpu.VMEM((N,), jnp.float32),
            pltpu.VMEM((N,), jnp.float32),
            pltpu.VMEM((N,), jnp.float32),
            [pltpu.VMEM((1, WIN), jnp.int32)] * 2,
            [pltpu.VMEM((1, WIN), jnp.int32)] * 2,
            [pltpu.VMEM((WIN, D), jnp.float32)] * 2,
            [pltpu.VMEM((WIN, D), jnp.float32)] * 2,
            [pltpu.VMEM((WIN,), jnp.float32)] * 2,
            [pltpu.VMEM((WIN,), jnp.float32)] * 2,
            [pltpu.VMEM((WIN,), jnp.float32)] * 2,
            [pltpu.VMEM((WIN,), jnp.float32)] * 2,
            [pltpu.SemaphoreType.DMA] * 2,
            [pltpu.SemaphoreType.DMA] * 2,
            [pltpu.SemaphoreType.DMA] * 2,
        ],
        compiler_params=_sc_params(),
    )
    def k(hp_h, hq_h, cx_h, cy_h, cz_h, gr_h, gc_h,
          a_h, b_h, rad_h, dx_h, dy_h, dz_h,
          cx_v, cy_v, cz_v, gr_v, gc_v, a_v, b_v,
          rad_v, dx_v, dy_v, dz_v, gi, gs, ws):
        c = lax.axis_index("c")
        s = lax.axis_index("s")
        w = s * NC + c
        base_g = w * WPT
        pltpu.sync_copy(cx_h, cx_v)
        pltpu.sync_copy(cy_h, cy_v)
        pltpu.sync_copy(cz_h, cz_v)

        def idx_copies(j, p, sem):
            return (pltpu.make_async_copy(gr_h.at[pl.ds(base_g + j, 1)], gr_v[p], sem),
                    pltpu.make_async_copy(gc_h.at[pl.ds(base_g + j, 1)], gc_v[p], sem))

        def gather_copies(p, sem):
            return (pltpu.make_async_copy(hp_h.at[gr_v[p].at[0]], a_v[p], sem),
                    pltpu.make_async_copy(hq_h.at[gc_v[p].at[0]], b_v[p], sem))

        def write_copies(j, p, sem):
            e0 = (base_g + j) * WIN
            return (pltpu.make_async_copy(a_v[p], a_h.at[pl.ds(e0, WIN)], sem),
                    pltpu.make_async_copy(b_v[p], b_h.at[pl.ds(e0, WIN)], sem),
                    pltpu.make_async_copy(rad_v[p], rad_h.at[pl.ds(e0, WIN)], sem),
                    pltpu.make_async_copy(dx_v[p], dx_h.at[pl.ds(e0, WIN)], sem),
                    pltpu.make_async_copy(dy_v[p], dy_h.at[pl.ds(e0, WIN)], sem),
                    pltpu.make_async_copy(dz_v[p], dz_h.at[pl.ds(e0, WIN)], sem))

        for cp in idx_copies(0, 0, gi[0]):
            cp.start()
        for cp in idx_copies(0, 0, gi[0]):
            cp.wait()
        for cp in idx_copies(1, 1, gi[1]):
            cp.start()
        for cp in gather_copies(0, gs[0]):
            cp.start()

        def step(j, p):
            q = 1 - p

            @pl.when(j > 0)
            def _():
                for cp in write_copies(j - 1, q, ws[q]):
                    cp.wait()

            @pl.when(j + 1 < WPT)
            def _():
                for cp in idx_copies(j + 1, q, gi[q]):
                    cp.wait()
                for cp in gather_copies(q, gs[q]):
                    cp.start()

            for cp in gather_copies(p, gs[p]):
                cp.wait()
            for g in range(WIN // 16):
                ir = gr_v[p][0, pl.ds(g * 16, 16)]
                ic = gc_v[p][0, pl.ds(g * 16, 16)]
                dx = plsc.load_gather(cx_v, [ir]) - plsc.load_gather(cx_v, [ic])
                dy = plsc.load_gather(cy_v, [ir]) - plsc.load_gather(cy_v, [ic])
                dz = plsc.load_gather(cz_v, [ir]) - plsc.load_gather(cz_v, [ic])
                dx_v[p][pl.ds(g * 16, 16)] = dx
                dy_v[p][pl.ds(g * 16, 16)] = dy
                dz_v[p][pl.ds(g * 16, 16)] = dz
                rad_v[p][pl.ds(g * 16, 16)] = dx * dx + dy * dy + dz * dz

            @pl.when(j + 2 < WPT)
            def _():
                for cp in idx_copies(j + 2, p, gi[p]):
                    cp.start()

            for cp in write_copies(j, p, ws[p]):
                cp.start()

        @pl.loop(0, WPT, step=2)
        def _(jj):
            step(jj, 0)
            step(jj + 1, 1)

        for cp in write_copies(WPT - 1, 1, ws[1]):
            cp.wait()

    return k(hp, hq, cx, cy, cz, g_row, g_col)


def _tc_edge_mlp(a, b, rad, dx, dy, dz, ea,
                 w_rad, w1e, b1, w2, b2, cw1, cb1, cw2, cb2):
    blk = 512

    def body(a_ref, b_ref, rad_ref, dx_ref, dy_ref, dz_ref, ea_ref,
             wr_ref, w1e_ref, b1_ref, w2_ref, b2_ref, cw1_ref, cb1_ref,
             cw2_ref, cb2_ref, m_ref, dxw_ref, dyw_ref, dzw_ref):
        rad_blk = rad_ref[...]
        x = (a_ref[...] + b_ref[...] + rad_blk * wr_ref[...]
             + jnp.dot(ea_ref[...], w1e_ref[...], preferred_element_type=jnp.float32)
             + b1_ref[...])
        t = _silu(x).astype(jnp.bfloat16)
        m = _silu(jnp.dot(t, w2_ref[...].astype(jnp.bfloat16),
                          preferred_element_type=jnp.float32)
                  + b2_ref[...])
        m_ref[...] = m
        u = _silu(jnp.dot(m.astype(jnp.bfloat16),
                          cw1_ref[...].astype(jnp.bfloat16),
                          preferred_element_type=jnp.float32)
                  + cb1_ref[...])
        wc = jnp.dot(u, cw2_ref[...], preferred_element_type=jnp.float32) + cb2_ref[...]
        wcn = wc / (jnp.sqrt(rad_blk) + 1e-8)
        dxw_ref[...] = dx_ref[...] * wcn
        dyw_ref[...] = dy_ref[...] * wcn
        dzw_ref[...] = dz_ref[...] * wcn

    full = lambda r, c: pl.BlockSpec((r, c), lambda i: (0, 0))
    col = pl.BlockSpec((blk, 1), lambda i: (i, 0))
    return pl.pallas_call(
        body,
        grid=(EP // blk,),
        in_specs=[
            pl.BlockSpec((blk, D), lambda i: (i, 0)),
            pl.BlockSpec((blk, D), lambda i: (i, 0)),
            col, col, col, col,
            pl.BlockSpec((blk, DE), lambda i: (i, 0)),
            full(1, D), full(DE, D), full(1, D), full(D, D), full(1, D),
            full(D, D), full(1, D), full(D, 1), full(1, 1),
        ],
        out_specs=[pl.BlockSpec((blk, D), lambda i: (i, 0)), col, col, col],
        out_shape=[
            jax.ShapeDtypeStruct((EP, D), jnp.float32),
            jax.ShapeDtypeStruct((EP, 1), jnp.float32),
            jax.ShapeDtypeStruct((EP, 1), jnp.float32),
            jax.ShapeDtypeStruct((EP, 1), jnp.float32),
        ],
    )(a, b, rad, dx, dy, dz, ea, w_rad, w1e, b1, w2, b2, cw1, cb1, cw2, cb2)


def _sc_scatter_m(m, s_row):
    mesh = plsc.VectorSubcoreMesh(core_axis_name="c", subcore_axis_name="s")
    out_type = jax.ShapeDtypeStruct((NC, NPAD, D), jnp.float32)

    @functools.partial(
        pl.kernel,
        out_type=out_type,
        mesh=mesh,
        scratch_types=[
            [pltpu.VMEM((1, WIN), jnp.int32)] * 2,
            [pltpu.VMEM((WIN, D), jnp.float32)] * 2,
            pltpu.VMEM_SHARED((NPAD, D), jnp.float32),
            [pltpu.SemaphoreType.DMA] * 2,
            [pltpu.SemaphoreType.DMA] * 2,
        ],
        compiler_params=_sc_params(),
    )
    def k(m_h, sr_h, mo_h, sr_v, m_v, macc, ls, ss):
        c = lax.axis_index("c")
        s = lax.axis_index("s")
        w = s * NC + c
        base_g = w * WPT
        zero16 = jnp.zeros((16,), jnp.float32)

        @pl.loop(0, WIN)
        def _(i):
            for g in range(D // 16):
                m_v[0][i, pl.ds(g * 16, 16)] = zero16

        row0 = s * RPT
        for kb in range(RPT // WIN):
            pltpu.sync_copy(m_v[0], macc.at[pl.ds(row0 + kb * WIN, WIN)])
        plsc.subcore_barrier()

        def load_copies(j, p, sem):
            gj = base_g + j
            return (pltpu.make_async_copy(sr_h.at[pl.ds(gj, 1)], sr_v[p], sem),
                    pltpu.make_async_copy(m_h.at[pl.ds(gj * WIN, WIN)], m_v[p], sem))

        for cp in load_copies(0, 0, ls[0]):
            cp.start()

        def step(j, p):
            q = 1 - p

            @pl.when(j + 1 < WPT)
            def _():
                for cp in load_copies(j + 1, q, ls[q]):
                    cp.start()

            for cp in load_copies(j, p, ls[p]):
                cp.wait()
            pltpu.sync_copy(m_v[p], macc.at[sr_v[p].at[0]], add=True)

        @pl.loop(0, WPT, step=2)
        def _(jj):
            step(jj, 0)
            step(jj + 1, 1)

        plsc.subcore_barrier()
        pltpu.sync_copy(macc.at[pl.ds(row0, RPT)], mo_h.at[c].at[pl.ds(row0, RPT)])

    return k(m, s_row)


def _sc_scatter_c(sr, dxw, dyw, dzw, z4):
    mesh = plsc.VectorSubcoreMesh(core_axis_name="c", subcore_axis_name="s")
    out_type = jax.ShapeDtypeStruct((NW, NPAD * 4), jnp.float32)

    @functools.partial(
        pl.kernel,
        out_type=out_type,
        mesh=mesh,
        scratch_types=[
            pltpu.VMEM((EPT,), jnp.int32),
            pltpu.VMEM((EPT,), jnp.float32),
            pltpu.VMEM((EPT,), jnp.float32),
            pltpu.VMEM((EPT,), jnp.float32),
            pltpu.VMEM((NPAD * 4,), jnp.float32),
        ],
        compiler_params=_sc_params(),
    )
    def k(sr_h, dx_h, dy_h, dz_h, z4_h, co_h, sr_v, dx_v, dy_v, dz_v, acc_v):
        c = lax.axis_index("c")
        s = lax.axis_index("s")
        w = s * NC + c
        e0 = w * EPT
        pltpu.sync_copy(z4_h, acc_v)
        pltpu.sync_copy(sr_h.at[pl.ds(e0, EPT)], sr_v)
        pltpu.sync_copy(dx_h.at[pl.ds(e0, EPT)], dx_v)
        pltpu.sync_copy(dy_h.at[pl.ds(e0, EPT)], dy_v)
        pltpu.sync_copy(dz_h.at[pl.ds(e0, EPT)], dz_v)
        @pl.loop(0, EPT // 16)
        def _(i):
            sl = pl.ds(i * 16, 16)
            idx = sr_v[sl] * 4
            plsc.addupdate_scatter(acc_v, [idx], dx_v[sl])
            plsc.addupdate_scatter(acc_v, [idx + 1], dy_v[sl])
            plsc.addupdate_scatter(acc_v, [idx + 2], dz_v[sl])

        pltpu.sync_copy(acc_v, co_h.at[w])

    return k(sr, dxw, dyw, dzw, z4)


def _tc_node(h, mparts, cparts, coord4, vel4,
             vw1, vb1, vw2, vb2, nw1a, nw1b, nb1, nw2, nb2):
    blk = 1000

    def body(h_ref, mp_ref, cp_ref, c4_ref, v4_ref, vw1_ref, vb1_ref, vw2_ref,
             vb2_ref, na_ref, nbm_ref, nb1_ref, nw2_ref, nb2_ref, ho_ref, co_ref):
        hh = h_ref[...]
        m_i = mp_ref[0] + mp_ref[1]
        csum = jnp.sum(cp_ref[...], axis=0)
        u = _silu(jnp.dot(hh, vw1_ref[...], preferred_element_type=jnp.float32)
                  + vb1_ref[...])
        vw = jnp.dot(u, vw2_ref[...], preferred_element_type=jnp.float32) + vb2_ref[...]
        co_ref[...] = c4_ref[...] + csum + v4_ref[...] * vw
        t = _silu(jnp.dot(hh, na_ref[...], preferred_element_type=jnp.float32)
                  + jnp.dot(m_i, nbm_ref[...], preferred_element_type=jnp.float32)
                  + nb1_ref[...])
        ho_ref[...] = jnp.dot(t, nw2_ref[...], preferred_element_type=jnp.float32) + nb2_ref[...]

    full = lambda r, c: pl.BlockSpec((r, c), lambda i: (0, 0))
    return pl.pallas_call(
        body,
        grid=(N // blk,),
        in_specs=[
            pl.BlockSpec((blk, D), lambda i: (i, 0)),
            pl.BlockSpec((NC, blk, D), lambda i: (0, i, 0)),
            pl.BlockSpec((NW, blk, 4), lambda i: (0, i, 0)),
            pl.BlockSpec((blk, 4), lambda i: (i, 0)),
            pl.BlockSpec((blk, 4), lambda i: (i, 0)),
            full(D, D), full(1, D), full(D, 1), full(1, 1),
            full(D, D), full(D, D), full(1, D), full(D, D), full(1, D),
        ],
        out_specs=[
            pl.BlockSpec((blk, D), lambda i: (i, 0)),
            pl.BlockSpec((blk, 4), lambda i: (i, 0)),
        ],
        out_shape=[
            jax.ShapeDtypeStruct((N, D), jnp.float32),
            jax.ShapeDtypeStruct((N, 4), jnp.float32),
        ],
    )(h, mparts, cparts, coord4, vel4, vw1, vb1, vw2, vb2, nw1a, nw1b, nb1, nw2, nb2)


def kernel(h, edge_index, coord, vel, edge_attr,
           edge_w1, edge_b1, edge_w2, edge_b2,
           coord_w1, coord_b1, coord_w2, coord_b2,
           vel_w1, vel_b1, vel_w2, vel_b2,
           node_w1, node_b1, node_w2, node_b2):
    w1a = edge_w1[:D]
    w1b = edge_w1[D:2 * D]
    w_rad = edge_w1[2 * D].reshape(1, D)
    w1e = edge_w1[2 * D + 1:]

    row = edge_index[0].astype(jnp.int32)
    col = edge_index[1].astype(jnp.int32)
    pad = EP - E
    g_row = jnp.concatenate([row, jnp.zeros((pad,), jnp.int32)]).reshape(NG, WIN)
    g_col = jnp.concatenate([col, jnp.zeros((pad,), jnp.int32)]).reshape(NG, WIN)
    s_row = jnp.concatenate([row, jnp.full((pad,), N, jnp.int32)]).reshape(NG, WIN)
    ea = jnp.concatenate([edge_attr, jnp.zeros((pad, DE), jnp.float32)], axis=0)

    cx = coord[:, 0] + 0.0
    cy = coord[:, 1] + 0.0
    cz = coord[:, 2] + 0.0
    coord4 = jnp.pad(coord, ((0, 0), (0, 1)))
    vel4 = jnp.pad(vel, ((0, 0), (0, 1)))
    z4 = jnp.zeros((NPAD * 4,), jnp.float32)

    hp, hq = _tc_project(h, w1a, w1b)
    a, b, rad, dx, dy, dz = _sc_gather(hp, hq, cx, cy, cz, g_row, g_col)
    m_ij, dxw, dyw, dzw = _tc_edge_mlp(
        a, b, rad.reshape(EP, 1), dx.reshape(EP, 1), dy.reshape(EP, 1),
        dz.reshape(EP, 1), ea, w_rad, w1e, edge_b1.reshape(1, D),
        edge_w2, edge_b2.reshape(1, D), coord_w1, coord_b1.reshape(1, D),
        coord_w2, coord_b2.reshape(1, 1))
    mparts = _sc_scatter_m(m_ij, s_row)
    cparts = _sc_scatter_c(s_row.reshape(EP), dxw.reshape(EP),
                           dyw.reshape(EP), dzw.reshape(EP), z4)
    cparts = cparts.reshape(NW, NPAD, 4)
    h_out, co4 = _tc_node(
        h, mparts, cparts, coord4, vel4,
        vel_w1, vel_b1.reshape(1, D), vel_w2, vel_b2.reshape(1, 1),
        node_w1[:D], node_w1[D:], node_b1.reshape(1, D),
        node_w2, node_b2.reshape(1, D))
    return (h_out, co4[:, :3])


# speedup vs baseline: 1.0689x; 1.0069x over previous
import dataclasses
import functools

import jax
import jax.numpy as jnp
from jax import lax
from jax.experimental import pallas as pl
from jax.experimental.pallas import tpu as pltpu
from jax.experimental.pallas import tpu_sc as plsc

N = 10000
D = 128
DE = 16
E = 320000
EP = 327680
WIN = 128
NG = EP // WIN
NC = 2
NS = 16
NW = NC * NS
WPT = NG // NW
EPT = EP // NW
NPAD = 10240
RPT = NPAD // NS
SEG = 16


def _sc_params():
    cp = pltpu.CompilerParams()
    if "needs_layout_passes" in pltpu.CompilerParams.__dataclass_fields__:
        cp = dataclasses.replace(cp, needs_layout_passes=False)
    return cp


def _silu(x):
    return x * jax.nn.sigmoid(x)


def _tc_project(h, w1a, w1b):
    blk = 1000

    def body(h_ref, wa_ref, wb_ref, hp_ref, hq_ref):
        hh = h_ref[...]
        hp_ref[...] = jnp.dot(hh, wa_ref[...], preferred_element_type=jnp.float32)
        hq_ref[...] = jnp.dot(hh, wb_ref[...], preferred_element_type=jnp.float32)

    return pl.pallas_call(
        body,
        grid=(N // blk,),
        in_specs=[
            pl.BlockSpec((blk, D), lambda i: (i, 0)),
            pl.BlockSpec((D, D), lambda i: (0, 0)),
            pl.BlockSpec((D, D), lambda i: (0, 0)),
        ],
        out_specs=[
            pl.BlockSpec((blk, D), lambda i: (i, 0)),
            pl.BlockSpec((blk, D), lambda i: (i, 0)),
        ],
        out_shape=[jax.ShapeDtypeStruct((N, D), jnp.float32)] * 2,
    )(h, w1a, w1b)


def _sc_gather(hp, hq, cx, cy, cz, g_row, g_col):
    mesh = plsc.VectorSubcoreMesh(core_axis_name="c", subcore_axis_name="s")
    out_type = (
        jax.ShapeDtypeStruct((EP, D), jnp.float32),
        jax.ShapeDtypeStruct((EP, D), jnp.float32),
        jax.ShapeDtypeStruct((EP,), jnp.float32),
        jax.ShapeDtypeStruct((EP,), jnp.float32),
        jax.ShapeDtypeStruct((EP,), jnp.float32),
        jax.ShapeDtypeStruct((EP,), jnp.float32),
    )

    @functools.partial(
        pl.kernel,
        out_type=out_type,
        mesh=mesh,
        scratch_types=[
            pltpu.VMEM((N,), jnp.float32),
            pltpu.VMEM((N,), jnp.float32),
            pltpu.VMEM((N,), jnp.float32),
            [pltpu.VMEM((1, WIN), jnp.int32)] * 2,
            [pltpu.VMEM((1, WIN), jnp.int32)] * 2,
            [pltpu.VMEM((WIN, D), jnp.float32)] * 2,
            [pltpu.VMEM((WIN, D), jnp.float32)] * 2,
            pltpu.VMEM((SEG * WIN,), jnp.float32),
            pltpu.VMEM((SEG * WIN,), jnp.float32),
            pltpu.VMEM((SEG * WIN,), jnp.float32),
            pltpu.VMEM((SEG * WIN,), jnp.float32),
            [pltpu.SemaphoreType.DMA] * 2,
            [pltpu.SemaphoreType.DMA] * 2,
            [pltpu.SemaphoreType.DMA] * 2,
        ],
        compiler_params=_sc_params(),
    )
    def k(hp_h, hq_h, cx_h, cy_h, cz_h, gr_h, gc_h,
          a_h, b_h, rad_h, dx_h, dy_h, dz_h,
          cx_v, cy_v, cz_v, gr_v, gc_v, a_v, b_v,
          rad_v, dx_v, dy_v, dz_v, gi, gs, ws):
        c = lax.axis_index("c")
        s = lax.axis_index("s")
        w = s * NC + c
        base_g = w * WPT
        pltpu.sync_copy(cx_h, cx_v)
        pltpu.sync_copy(cy_h, cy_v)
        pltpu.sync_copy(cz_h, cz_v)

        def idx_copies(j, p, sem):
            return (pltpu.make_async_copy(gr_h.at[pl.ds(base_g + j, 1)], gr_v[p], sem),
                    pltpu.make_async_copy(gc_h.at[pl.ds(base_g + j, 1)], gc_v[p], sem))

        def gather_copies(p, sem):
            return (pltpu.make_async_copy(hp_h.at[gr_v[p].at[0]], a_v[p], sem),
                    pltpu.make_async_copy(hq_h.at[gc_v[p].at[0]], b_v[p], sem))

        def write_copies(j, p, sem):
            e0 = (base_g + j) * WIN
            return (pltpu.make_async_copy(a_v[p], a_h.at[pl.ds(e0, WIN)], sem),
                    pltpu.make_async_copy(b_v[p], b_h.at[pl.ds(e0, WIN)], sem))

        for cp in idx_copies(0, 0, gi[0]):
            cp.start()
        for cp in idx_copies(0, 0, gi[0]):
            cp.wait()
        for cp in idx_copies(1, 1, gi[1]):
            cp.start()
        for cp in gather_copies(0, gs[0]):
            cp.start()

        def step(j, p):
            q = 1 - p

            @pl.when(j > 0)
            def _():
                for cp in write_copies(j - 1, q, ws[q]):
                    cp.wait()

            @pl.when(j + 1 < WPT)
            def _():
                for cp in idx_copies(j + 1, q, gi[q]):
                    cp.wait()
                for cp in gather_copies(q, gs[q]):
                    cp.start()

            for cp in gather_copies(p, gs[p]):
                cp.wait()
            for g in range(WIN // 16):
                ir = gr_v[p][0, pl.ds(g * 16, 16)]
                ic = gc_v[p][0, pl.ds(g * 16, 16)]
                dx = plsc.load_gather(cx_v, [ir]) - plsc.load_gather(cx_v, [ic])
                dy = plsc.load_gather(cy_v, [ir]) - plsc.load_gather(cy_v, [ic])
                dz = plsc.load_gather(cz_v, [ir]) - plsc.load_gather(cz_v, [ic])
                sl = pl.ds((j % SEG) * WIN + g * 16, 16)
                dx_v[sl] = dx
                dy_v[sl] = dy
                dz_v[sl] = dz
                rad_v[sl] = dx * dx + dy * dy + dz * dz

            @pl.when(j % SEG == SEG - 1)
            def _():
                f0 = w * EPT + (j - (SEG - 1)) * WIN
                pltpu.sync_copy(rad_v, rad_h.at[pl.ds(f0, SEG * WIN)])
                pltpu.sync_copy(dx_v, dx_h.at[pl.ds(f0, SEG * WIN)])
                pltpu.sync_copy(dy_v, dy_h.at[pl.ds(f0, SEG * WIN)])
                pltpu.sync_copy(dz_v, dz_h.at[pl.ds(f0, SEG * WIN)])

            @pl.when(j + 2 < WPT)
            def _():
                for cp in idx_copies(j + 2, p, gi[p]):
                    cp.start()

            for cp in write_copies(j, p, ws[p]):
                cp.start()

        @pl.loop(0, WPT, step=2)
        def _(jj):
            step(jj, 0)
            step(jj + 1, 1)

        for cp in write_copies(WPT - 1, 1, ws[1]):
            cp.wait()

    return k(hp, hq, cx, cy, cz, g_row, g_col)


def _tc_edge_mlp(a, b, rad, dx, dy, dz, ea,
                 w_rad, w1e, b1, w2, b2, cw1, cb1, cw2, cb2):
    blk = 512

    def body(a_ref, b_ref, rad_ref, dx_ref, dy_ref, dz_ref, ea_ref,
             wr_ref, w1e_ref, b1_ref, w2_ref, b2_ref, cw1_ref, cb1_ref,
             cw2_ref, cb2_ref, m_ref, dxw_ref, dyw_ref, dzw_ref):
        rad_blk = rad_ref[...]
        x = (a_ref[...].astype(jnp.float32) + b_ref[...].astype(jnp.float32)
             + rad_blk * wr_ref[...]
             + jnp.dot(ea_ref[...], w1e_ref[...], preferred_element_type=jnp.float32)
             + b1_ref[...])
        t = _silu(x).astype(jnp.bfloat16)
        m = _silu(jnp.dot(t, w2_ref[...].astype(jnp.bfloat16),
                          preferred_element_type=jnp.float32)
                  + b2_ref[...])
        m_ref[...] = m
        u = _silu(jnp.dot(m.astype(jnp.bfloat16),
                          cw1_ref[...].astype(jnp.bfloat16),
                          preferred_element_type=jnp.float32)
                  + cb1_ref[...])
        wc = jnp.dot(u, cw2_ref[...], preferred_element_type=jnp.float32) + cb2_ref[...]
        wcn = wc / (jnp.sqrt(rad_blk) + 1e-8)
        dxw_ref[...] = dx_ref[...] * wcn
        dyw_ref[...] = dy_ref[...] * wcn
        dzw_ref[...] = dz_ref[...] * wcn

    full = lambda r, c: pl.BlockSpec((r, c), lambda i: (0, 0))
    col = pl.BlockSpec((blk, 1), lambda i: (i, 0))
    return pl.pallas_call(
        body,
        grid=(EP // blk,),
        in_specs=[
            pl.BlockSpec((blk, D), lambda i: (i, 0)),
            pl.BlockSpec((blk, D), lambda i: (i, 0)),
            col, col, col, col,
            pl.BlockSpec((blk, DE), lambda i: (i, 0)),
            full(1, D), full(DE, D), full(1, D), full(D, D), full(1, D),
            full(D, D), full(1, D), full(D, 1), full(1, 1),
        ],
        out_specs=[pl.BlockSpec((blk, D), lambda i: (i, 0)), col, col, col],
        out_shape=[
            jax.ShapeDtypeStruct((EP, D), jnp.float32),
            jax.ShapeDtypeStruct((EP, 1), jnp.float32),
            jax.ShapeDtypeStruct((EP, 1), jnp.float32),
            jax.ShapeDtypeStruct((EP, 1), jnp.float32),
        ],
    )(a, b, rad, dx, dy, dz, ea, w_rad, w1e, b1, w2, b2, cw1, cb1, cw2, cb2)


def _sc_scatter_m(m, s_row):
    mesh = plsc.VectorSubcoreMesh(core_axis_name="c", subcore_axis_name="s")
    out_type = jax.ShapeDtypeStruct((NC, NPAD, D), jnp.float32)

    @functools.partial(
        pl.kernel,
        out_type=out_type,
        mesh=mesh,
        scratch_types=[
            [pltpu.VMEM((1, WIN), jnp.int32)] * 2,
            [pltpu.VMEM((WIN, D), jnp.float32)] * 2,
            pltpu.VMEM_SHARED((NPAD, D), jnp.float32),
            [pltpu.SemaphoreType.DMA] * 2,
            [pltpu.SemaphoreType.DMA] * 2,
        ],
        compiler_params=_sc_params(),
    )
    def k(m_h, sr_h, mo_h, sr_v, m_v, macc, ls, ss):
        c = lax.axis_index("c")
        s = lax.axis_index("s")
        w = s * NC + c
        base_g = w * WPT
        zero16 = jnp.zeros((16,), jnp.float32)

        @pl.loop(0, WIN)
        def _(i):
            for g in range(D // 16):
                m_v[0][i, pl.ds(g * 16, 16)] = zero16

        row0 = s * RPT
        for kb in range(RPT // WIN):
            pltpu.sync_copy(m_v[0], macc.at[pl.ds(row0 + kb * WIN, WIN)])
        plsc.subcore_barrier()

        def load_copies(j, p, sem):
            gj = base_g + j
            return (pltpu.make_async_copy(sr_h.at[pl.ds(gj, 1)], sr_v[p], sem),
                    pltpu.make_async_copy(m_h.at[pl.ds(gj * WIN, WIN)], m_v[p], sem))

        for cp in load_copies(0, 0, ls[0]):
            cp.start()

        def step(j, p):
            q = 1 - p

            @pl.when(j + 1 < WPT)
            def _():
                for cp in load_copies(j + 1, q, ls[q]):
                    cp.start()

            for cp in load_copies(j, p, ls[p]):
                cp.wait()
            pltpu.sync_copy(m_v[p], macc.at[sr_v[p].at[0]], add=True)

        @pl.loop(0, WPT, step=2)
        def _(jj):
            step(jj, 0)
            step(jj + 1, 1)

        plsc.subcore_barrier()
        pltpu.sync_copy(macc.at[pl.ds(row0, RPT)], mo_h.at[c].at[pl.ds(row0, RPT)])

    return k(m, s_row)


def _sc_scatter_c(sr, dxw, dyw, dzw, z4):
    mesh = plsc.VectorSubcoreMesh(core_axis_name="c", subcore_axis_name="s")
    out_type = jax.ShapeDtypeStruct((NW, NPAD * 4), jnp.float32)

    @functools.partial(
        pl.kernel,
        out_type=out_type,
        mesh=mesh,
        scratch_types=[
            pltpu.VMEM((EPT,), jnp.int32),
            pltpu.VMEM((EPT,), jnp.float32),
            pltpu.VMEM((EPT,), jnp.float32),
            pltpu.VMEM((EPT,), jnp.float32),
            pltpu.VMEM((NPAD * 4,), jnp.float32),
        ],
        compiler_params=_sc_params(),
    )
    def k(sr_h, dx_h, dy_h, dz_h, z4_h, co_h, sr_v, dx_v, dy_v, dz_v, acc_v):
        c = lax.axis_index("c")
        s = lax.axis_index("s")
        w = s * NC + c
        e0 = w * EPT
        pltpu.sync_copy(z4_h, acc_v)
        pltpu.sync_copy(sr_h.at[pl.ds(e0, EPT)], sr_v)
        pltpu.sync_copy(dx_h.at[pl.ds(e0, EPT)], dx_v)
        pltpu.sync_copy(dy_h.at[pl.ds(e0, EPT)], dy_v)
        pltpu.sync_copy(dz_h.at[pl.ds(e0, EPT)], dz_v)
        @pl.loop(0, EPT // 16)
        def _(i):
            sl = pl.ds(i * 16, 16)
            idx = sr_v[sl] * 4
            plsc.addupdate_scatter(acc_v, [idx], dx_v[sl])
            plsc.addupdate_scatter(acc_v, [idx + 1], dy_v[sl])
            plsc.addupdate_scatter(acc_v, [idx + 2], dz_v[sl])

        pltpu.sync_copy(acc_v, co_h.at[w])

    return k(sr, dxw, dyw, dzw, z4)


def _tc_node(h, mparts, cparts, coord4, vel4,
             vw1, vb1, vw2, vb2, nw1a, nw1b, nb1, nw2, nb2):
    blk = 1000

    def body(h_ref, mp_ref, cp_ref, c4_ref, v4_ref, vw1_ref, vb1_ref, vw2_ref,
             vb2_ref, na_ref, nbm_ref, nb1_ref, nw2_ref, nb2_ref, ho_ref, co_ref):
        hh = h_ref[...]
        m_i = mp_ref[0] + mp_ref[1]
        csum = jnp.sum(cp_ref[...], axis=0)
        u = _silu(jnp.dot(hh, vw1_ref[...], preferred_element_type=jnp.float32)
                  + vb1_ref[...])
        vw = jnp.dot(u, vw2_ref[...], preferred_element_type=jnp.float32) + vb2_ref[...]
        co_ref[...] = c4_ref[...] + csum + v4_ref[...] * vw
        t = _silu(jnp.dot(hh, na_ref[...], preferred_element_type=jnp.float32)
                  + jnp.dot(m_i, nbm_ref[...], preferred_element_type=jnp.float32)
                  + nb1_ref[...])
        ho_ref[...] = jnp.dot(t, nw2_ref[...], preferred_element_type=jnp.float32) + nb2_ref[...]

    full = lambda r, c: pl.BlockSpec((r, c), lambda i: (0, 0))
    return pl.pallas_call(
        body,
        grid=(N // blk,),
        in_specs=[
            pl.BlockSpec((blk, D), lambda i: (i, 0)),
            pl.BlockSpec((NC, blk, D), lambda i: (0, i, 0)),
            pl.BlockSpec((NW, blk, 4), lambda i: (0, i, 0)),
            pl.BlockSpec((blk, 4), lambda i: (i, 0)),
            pl.BlockSpec((blk, 4), lambda i: (i, 0)),
            full(D, D), full(1, D), full(D, 1), full(1, 1),
            full(D, D), full(D, D), full(1, D), full(D, D), full(1, D),
        ],
        out_specs=[
            pl.BlockSpec((blk, D), lambda i: (i, 0)),
            pl.BlockSpec((blk, 4), lambda i: (i, 0)),
        ],
        out_shape=[
            jax.ShapeDtypeStruct((N, D), jnp.float32),
            jax.ShapeDtypeStruct((N, 4), jnp.float32),
        ],
    )(h, mparts, cparts, coord4, vel4, vw1, vb1, vw2, vb2, nw1a, nw1b, nb1, nw2, nb2)


def kernel(h, edge_index, coord, vel, edge_attr,
           edge_w1, edge_b1, edge_w2, edge_b2,
           coord_w1, coord_b1, coord_w2, coord_b2,
           vel_w1, vel_b1, vel_w2, vel_b2,
           node_w1, node_b1, node_w2, node_b2):
    w1a = edge_w1[:D]
    w1b = edge_w1[D:2 * D]
    w_rad = edge_w1[2 * D].reshape(1, D)
    w1e = edge_w1[2 * D + 1:]

    row = edge_index[0].astype(jnp.int32)
    col = edge_index[1].astype(jnp.int32)
    pad = EP - E
    g_row = jnp.concatenate([row, jnp.zeros((pad,), jnp.int32)]).reshape(NG, WIN)
    g_col = jnp.concatenate([col, jnp.zeros((pad,), jnp.int32)]).reshape(NG, WIN)
    s_row = jnp.concatenate([row, jnp.full((pad,), N, jnp.int32)]).reshape(NG, WIN)
    ea = jnp.concatenate([edge_attr, jnp.zeros((pad, DE), jnp.float32)], axis=0)

    cx = coord[:, 0] + 0.0
    cy = coord[:, 1] + 0.0
    cz = coord[:, 2] + 0.0
    coord4 = jnp.pad(coord, ((0, 0), (0, 1)))
    vel4 = jnp.pad(vel, ((0, 0), (0, 1)))
    z4 = jnp.zeros((NPAD * 4,), jnp.float32)

    hp, hq = _tc_project(h, w1a, w1b)
    a, b, rad, dx, dy, dz = _sc_gather(hp, hq, cx, cy, cz, g_row, g_col)
    m_ij, dxw, dyw, dzw = _tc_edge_mlp(
        a, b, rad.reshape(EP, 1), dx.reshape(EP, 1), dy.reshape(EP, 1),
        dz.reshape(EP, 1), ea, w_rad, w1e, edge_b1.reshape(1, D),
        edge_w2, edge_b2.reshape(1, D), coord_w1, coord_b1.reshape(1, D),
        coord_w2, coord_b2.reshape(1, 1))
    mparts = _sc_scatter_m(m_ij, s_row)
    cparts = _sc_scatter_c(s_row.reshape(EP), dxw.reshape(EP),
                           dyw.reshape(EP), dzw.reshape(EP), z4)
    cparts = cparts.reshape(NW, NPAD, 4)
    h_out, co4 = _tc_node(
        h, mparts, cparts, coord4, vel4,
        vel_w1, vel_b1.reshape(1, D), vel_w2, vel_b2.reshape(1, 1),
        node_w1[:D], node_w1[D:], node_b1.reshape(1, D),
        node_w2, node_b2.reshape(1, D))
    return (h_out, co4[:, :3])
